# Initial kernel scaffold; baseline (speedup 1.0000x reference)
import functools

import jax
import jax.numpy as jnp
from jax import lax
from jax.experimental import pallas as pl
from jax.experimental.pallas import tpu as pltpu
from jax.experimental.pallas import tpu_sc as plsc

_LANES = 16


@functools.lru_cache(maxsize=None)
def _build_sc_kernel(S, D, T, NC, NS, JC):
    NW = NC * NS
    ROWS = S // NW
    NJC = S // JC
    VPR = D // _LANES

    mesh = plsc.VectorSubcoreMesh(core_axis_name="c", subcore_axis_name="s")

    @functools.partial(
        pl.kernel,
        out_type=jax.ShapeDtypeStruct((S, S, D), jnp.float32),
        mesh=mesh,
        scratch_types=[
            pltpu.VMEM((JC, D), jnp.float32),
            pltpu.VMEM((JC, D), jnp.float32),
            pltpu.VMEM((JC, D), jnp.float32),
        ],
    )
    def sc_kernel(x_hbm, tab_hbm, out_hbm, xbuf, tbuf, obuf):
        wid = lax.axis_index("s") * NC + lax.axis_index("c")
        i0 = wid * ROWS

        def jloop(jc, _):
            j0 = jc * JC
            pltpu.sync_copy(x_hbm.at[pl.ds(j0, JC)], xbuf)

            def rloop(r, _):
                i = i0 + r
                start = i + (S - JC + 1) - j0
                pltpu.sync_copy(tab_hbm.at[pl.ds(start, JC)], tbuf)

                def cloop(jj, _):
                    rj = JC - 1 - jj
                    for v in range(VPR):
                        sl = pl.ds(v * _LANES, _LANES)
                        obuf[jj, sl] = xbuf[jj, sl] + tbuf[rj, sl]
                    return 0

                lax.fori_loop(0, JC, cloop, 0)
                pltpu.sync_copy(obuf, out_hbm.at[i, pl.ds(j0, JC)])
                return 0

            lax.fori_loop(0, ROWS, rloop, 0)
            return 0

        lax.fori_loop(0, NJC, jloop, 0)

    return sc_kernel


def kernel(x, rel_pos_embeddings):
    batch, S, D = x.shape
    T = rel_pos_embeddings.shape[0]
    info = plsc.get_sparse_core_info()
    sc = _build_sc_kernel(S, D, T, info.num_cores, info.num_subcores, 128)
    return sc(x.reshape(S, D), rel_pos_embeddings)


# speedup vs baseline: 2.0934x; 2.0934x over previous
import functools

import jax
import jax.numpy as jnp
from jax import lax
from jax.experimental import pallas as pl
from jax.experimental.pallas import tpu as pltpu
from jax.experimental.pallas import tpu_sc as plsc

_LANES = 16


@functools.lru_cache(maxsize=None)
def _build_sc_kernel(S, D, T, NC, NS, JC):
    NW = NC * NS
    ROWS = S // NW
    NJC = S // JC
    VPR = D // _LANES

    mesh = plsc.VectorSubcoreMesh(core_axis_name="c", subcore_axis_name="s")

    @functools.partial(
        pl.kernel,
        out_type=jax.ShapeDtypeStruct((S, S, D), jnp.float32),
        mesh=mesh,
        scratch_types=[
            pltpu.VMEM((JC, D), jnp.float32),
            pltpu.VMEM((JC * D,), jnp.float32),
            pltpu.VMEM((JC, D), jnp.float32),
        ],
    )
    def sc_kernel(x_hbm, tab_hbm, out_hbm, xbuf, tbuf, obuf):
        wid = lax.axis_index("s") * NC + lax.axis_index("c")
        i0 = wid * ROWS
        ML = (T - 1) // 2

        def jloop(jc, _):
            j0 = jc * JC
            pltpu.sync_copy(x_hbm.at[pl.ds(j0, JC)], xbuf)

            def rloop(r, _):
                i = i0 + r
                start = i + (ML - JC + 1) - j0
                pltpu.sync_copy(tab_hbm.at[pl.ds(start * D, JC * D)], tbuf)

                def cloop(jj, _):
                    rj = JC - 1 - jj
                    for v in range(VPR):
                        sl = pl.ds(v * _LANES, _LANES)
                        obuf[jj, sl] = xbuf[jj, sl] + tbuf[pl.ds(rj * D + v * _LANES, _LANES)]
                    return 0

                lax.fori_loop(0, JC, cloop, 0)
                pltpu.sync_copy(obuf, out_hbm.at[i, pl.ds(j0, JC)])
                return 0

            lax.fori_loop(0, ROWS, rloop, 0)
            return 0

        lax.fori_loop(0, NJC, jloop, 0)

    return sc_kernel


def kernel(x, rel_pos_embeddings):
    batch, S, D = x.shape
    T = rel_pos_embeddings.shape[0]
    info = plsc.get_sparse_core_info()
    sc = _build_sc_kernel(S, D, T, info.num_cores, info.num_subcores, 128)
    return sc(x.reshape(S, D), rel_pos_embeddings.reshape(T * D))


# speedup vs baseline: 2.9888x; 1.4277x over previous
import functools

import jax
import jax.numpy as jnp
from jax import lax
from jax.experimental import pallas as pl
from jax.experimental.pallas import tpu as pltpu
from jax.experimental.pallas import tpu_sc as plsc

_LANES = 16


@functools.lru_cache(maxsize=None)
def _build_sc_kernel(S, D, T, NC, NS, JC):
    NW = NC * NS
    ROWS = S // NW
    NJC = S // JC
    VPR = D // _LANES

    mesh = plsc.VectorSubcoreMesh(core_axis_name="c", subcore_axis_name="s")

    HALF = ROWS // 2

    @functools.partial(
        pl.kernel,
        out_type=jax.ShapeDtypeStruct((S, S, D), jnp.float32),
        mesh=mesh,
        scratch_types=[
            pltpu.VMEM((JC, D), jnp.float32),
            pltpu.VMEM((JC * D,), jnp.float32),
            pltpu.VMEM((JC * D,), jnp.float32),
            pltpu.VMEM((JC, D), jnp.float32),
            pltpu.VMEM((JC, D), jnp.float32),
            pltpu.SemaphoreType.DMA,
            pltpu.SemaphoreType.DMA,
            pltpu.SemaphoreType.DMA,
            pltpu.SemaphoreType.DMA,
        ],
    )
    def sc_kernel(x_hbm, tab_hbm, out_hbm, xbuf, tb0, tb1, ob0, ob1,
                  tsem0, tsem1, osem0, osem1):
        wid = lax.axis_index("s") * NC + lax.axis_index("c")
        i0 = wid * ROWS
        ML = (T - 1) // 2

        def tstart(i, j0, tb, sem):
            start = i + (ML - JC + 1) - j0
            pltpu.make_async_copy(
                tab_hbm.at[pl.ds(start * D, JC * D)], tb, sem).start()

        def twait(tb, sem):
            pltpu.make_async_copy(tab_hbm.at[pl.ds(0, JC * D)], tb, sem).wait()

        def ostart(i, j0, ob, sem):
            pltpu.make_async_copy(ob, out_hbm.at[i, pl.ds(j0, JC)], sem).start()

        def owait(ob, sem):
            pltpu.make_async_copy(out_hbm.at[0, pl.ds(0, JC)], ob, sem).wait()

        def compute(tb, ob):
            def cloop(jj, _):
                rj = JC - 1 - jj
                for v in range(VPR):
                    sl = pl.ds(v * _LANES, _LANES)
                    ob[jj, sl] = xbuf[jj, sl] + tb[pl.ds(rj * D + v * _LANES, _LANES)]
                return 0
            lax.fori_loop(0, JC, cloop, 0)

        def jloop(jc, _):
            j0 = jc * JC
            pltpu.sync_copy(x_hbm.at[pl.ds(j0, JC)], xbuf)
            tstart(i0, j0, tb0, tsem0)

            def tloop(t, _):
                r0 = 2 * t
                r1 = r0 + 1
                tstart(i0 + r1, j0, tb1, tsem1)
                twait(tb0, tsem0)

                @pl.when(t > 0)
                def _():
                    owait(ob0, osem0)

                compute(tb0, ob0)
                ostart(i0 + r0, j0, ob0, osem0)

                @pl.when(t < HALF - 1)
                def _():
                    tstart(i0 + r1 + 1, j0, tb0, tsem0)

                twait(tb1, tsem1)

                @pl.when(t > 0)
                def _():
                    owait(ob1, osem1)

                compute(tb1, ob1)
                ostart(i0 + r1, j0, ob1, osem1)
                return 0

            lax.fori_loop(0, HALF, tloop, 0)
            owait(ob0, osem0)
            owait(ob1, osem1)
            return 0

        lax.fori_loop(0, NJC, jloop, 0)

    return sc_kernel


def kernel(x, rel_pos_embeddings):
    batch, S, D = x.shape
    T = rel_pos_embeddings.shape[0]
    info = plsc.get_sparse_core_info()
    sc = _build_sc_kernel(S, D, T, info.num_cores, info.num_subcores, 128)
    return sc(x.reshape(S, D), rel_pos_embeddings.reshape(T * D))


# speedup vs baseline: 5.7931x; 1.9383x over previous
import functools

import jax
import jax.numpy as jnp
from jax import lax
from jax.experimental import pallas as pl
from jax.experimental.pallas import tpu as pltpu
from jax.experimental.pallas import tpu_sc as plsc

_LANES = 16


@functools.lru_cache(maxsize=None)
def _build_sc_kernel(S, D, T, NC, NS, JC):
    NW = NC * NS
    ROWS = S // NW
    NJC = S // JC
    VPR = D // _LANES

    mesh = plsc.VectorSubcoreMesh(core_axis_name="c", subcore_axis_name="s")

    HALF = ROWS // 2

    @functools.partial(
        pl.kernel,
        out_type=jax.ShapeDtypeStruct((S, S, D), jnp.float32),
        mesh=mesh,
        scratch_types=[
            pltpu.VMEM((JC, D), jnp.float32),
            pltpu.VMEM((JC * D,), jnp.float32),
            pltpu.VMEM((JC * D,), jnp.float32),
            pltpu.VMEM((JC, D), jnp.float32),
            pltpu.VMEM((JC, D), jnp.float32),
            pltpu.SemaphoreType.DMA,
            pltpu.SemaphoreType.DMA,
            pltpu.SemaphoreType.DMA,
            pltpu.SemaphoreType.DMA,
        ],
    )
    def sc_kernel(x_hbm, tab_hbm, out_hbm, xbuf, tb0, tb1, ob0, ob1,
                  tsem0, tsem1, osem0, osem1):
        wid = lax.axis_index("s") * NC + lax.axis_index("c")
        i0 = wid * ROWS
        ML = (T - 1) // 2

        def tstart(i, j0, tb, sem):
            start = i + (ML - JC + 1) - j0
            pltpu.make_async_copy(
                tab_hbm.at[pl.ds(start * D, JC * D)], tb, sem).start()

        def twait(tb, sem):
            pltpu.make_async_copy(tab_hbm.at[pl.ds(0, JC * D)], tb, sem).wait()

        def ostart(i, j0, ob, sem):
            pltpu.make_async_copy(ob, out_hbm.at[i, pl.ds(j0, JC)], sem).start()

        def owait(ob, sem):
            pltpu.make_async_copy(out_hbm.at[0, pl.ds(0, JC)], ob, sem).wait()

        def compute(tb, ob):
            @plsc.parallel_loop(0, JC, unroll=4)
            def _(jj):
                rj = JC - 1 - jj
                for v in range(VPR):
                    sl = pl.ds(v * _LANES, _LANES)
                    ob[jj, sl] = xbuf[jj, sl] + tb[pl.ds(rj * D + v * _LANES, _LANES)]

        def jloop(jc, _):
            j0 = jc * JC
            pltpu.sync_copy(x_hbm.at[pl.ds(j0, JC)], xbuf)
            tstart(i0, j0, tb0, tsem0)

            def tloop(t, _):
                r0 = 2 * t
                r1 = r0 + 1
                tstart(i0 + r1, j0, tb1, tsem1)
                twait(tb0, tsem0)

                @pl.when(t > 0)
                def _():
                    owait(ob0, osem0)

                compute(tb0, ob0)
                ostart(i0 + r0, j0, ob0, osem0)

                @pl.when(t < HALF - 1)
                def _():
                    tstart(i0 + r1 + 1, j0, tb0, tsem0)

                twait(tb1, tsem1)

                @pl.when(t > 0)
                def _():
                    owait(ob1, osem1)

                compute(tb1, ob1)
                ostart(i0 + r1, j0, ob1, osem1)
                return 0

            lax.fori_loop(0, HALF, tloop, 0)
            owait(ob0, osem0)
            owait(ob1, osem1)
            return 0

        lax.fori_loop(0, NJC, jloop, 0)

    return sc_kernel


def kernel(x, rel_pos_embeddings):
    batch, S, D = x.shape
    T = rel_pos_embeddings.shape[0]
    info = plsc.get_sparse_core_info()
    sc = _build_sc_kernel(S, D, T, info.num_cores, info.num_subcores, 128)
    return sc(x.reshape(S, D), rel_pos_embeddings.reshape(T * D))


# speedup vs baseline: 5.7962x; 1.0005x over previous
import functools

import jax
import jax.numpy as jnp
from jax import lax
from jax.experimental import pallas as pl
from jax.experimental.pallas import tpu as pltpu
from jax.experimental.pallas import tpu_sc as plsc

_LANES = 16


@functools.lru_cache(maxsize=None)
def _build_sc_kernel(S, D, T, NC, NS, JC):
    NW = NC * NS
    ROWS = S // NW
    NJC = S // JC
    VPR = D // _LANES

    mesh = plsc.VectorSubcoreMesh(core_axis_name="c", subcore_axis_name="s")

    HALF = ROWS // 2

    @functools.partial(
        pl.kernel,
        out_type=jax.ShapeDtypeStruct((S, S, D), jnp.float32),
        mesh=mesh,
        scratch_types=[
            pltpu.VMEM((JC, D), jnp.float32),
            pltpu.VMEM((JC * D,), jnp.float32),
            pltpu.VMEM((JC * D,), jnp.float32),
            pltpu.VMEM((JC, D), jnp.float32),
            pltpu.VMEM((JC, D), jnp.float32),
            pltpu.SemaphoreType.DMA,
            pltpu.SemaphoreType.DMA,
            pltpu.SemaphoreType.DMA,
            pltpu.SemaphoreType.DMA,
        ],
    )
    def sc_kernel(x_hbm, tab_hbm, out_hbm, xbuf, tb0, tb1, ob0, ob1,
                  tsem0, tsem1, osem0, osem1):
        wid = lax.axis_index("s") * NC + lax.axis_index("c")
        i0 = wid * ROWS
        ML = (T - 1) // 2

        def tstart(i, j0, tb, sem):
            start = i + (ML - JC + 1) - j0
            pltpu.make_async_copy(
                tab_hbm.at[pl.ds(start * D, JC * D)], tb, sem).start()

        def twait(tb, sem):
            pltpu.make_async_copy(tab_hbm.at[pl.ds(0, JC * D)], tb, sem).wait()

        def ostart(i, j0, ob, sem):
            pltpu.make_async_copy(ob, out_hbm.at[i, pl.ds(j0, JC)], sem).start()

        def owait(ob, sem):
            pltpu.make_async_copy(out_hbm.at[0, pl.ds(0, JC)], ob, sem).wait()

        def compute(tb, ob):
            @plsc.parallel_loop(0, JC, unroll=8)
            def _(jj):
                rj = JC - 1 - jj
                for v in range(VPR):
                    sl = pl.ds(v * _LANES, _LANES)
                    ob[jj, sl] = xbuf[jj, sl] + tb[pl.ds(rj * D + v * _LANES, _LANES)]

        def jloop(jc, _):
            j0 = jc * JC
            pltpu.sync_copy(x_hbm.at[pl.ds(j0, JC)], xbuf)
            tstart(i0, j0, tb0, tsem0)

            def tloop(t, _):
                r0 = 2 * t
                r1 = r0 + 1
                tstart(i0 + r1, j0, tb1, tsem1)
                twait(tb0, tsem0)

                @pl.when(t > 0)
                def _():
                    owait(ob0, osem0)

                compute(tb0, ob0)
                ostart(i0 + r0, j0, ob0, osem0)

                @pl.when(t < HALF - 1)
                def _():
                    tstart(i0 + r1 + 1, j0, tb0, tsem0)

                twait(tb1, tsem1)

                @pl.when(t > 0)
                def _():
                    owait(ob1, osem1)

                compute(tb1, ob1)
                ostart(i0 + r1, j0, ob1, osem1)
                return 0

            lax.fori_loop(0, HALF, tloop, 0)
            owait(ob0, osem0)
            owait(ob1, osem1)
            return 0

        lax.fori_loop(0, NJC, jloop, 0)

    return sc_kernel


def kernel(x, rel_pos_embeddings):
    batch, S, D = x.shape
    T = rel_pos_embeddings.shape[0]
    info = plsc.get_sparse_core_info()
    sc = _build_sc_kernel(S, D, T, info.num_cores, info.num_subcores, 128)
    return sc(x.reshape(S, D), rel_pos_embeddings.reshape(T * D))


# speedup vs baseline: 5.8582x; 1.0107x over previous
import functools

import jax
import jax.numpy as jnp
from jax import lax
from jax.experimental import pallas as pl
from jax.experimental.pallas import tpu as pltpu
from jax.experimental.pallas import tpu_sc as plsc

_LANES = 16


@functools.lru_cache(maxsize=None)
def _build_sc_kernel(S, D, T, NC, NS, JC):
    NW = NC * NS
    ROWS = S // NW
    NJC = S // JC
    VPR = D // _LANES

    mesh = plsc.VectorSubcoreMesh(core_axis_name="c", subcore_axis_name="s")

    HALF = ROWS // 2
    QT = HALF // 2

    @functools.partial(
        pl.kernel,
        out_type=jax.ShapeDtypeStruct((S, S, D), jnp.float32),
        mesh=mesh,
        scratch_types=[
            pltpu.VMEM((JC, D), jnp.float32),
            [pltpu.VMEM((JC * D,), jnp.float32) for _ in range(4)],
            [pltpu.VMEM((JC, D), jnp.float32) for _ in range(4)],
            [pltpu.SemaphoreType.DMA for _ in range(8)],
        ],
    )
    def sc_kernel(x_hbm, tab_hbm, out_hbm, xbuf, tbs, obs, sems):
        tsems, osems = sems[:4], sems[4:]
        wid = lax.axis_index("s") * NC + lax.axis_index("c")
        i0 = wid * ROWS
        ML = (T - 1) // 2

        def tstart(i, j0, b):
            start = i + (ML - JC + 1) - j0
            pltpu.make_async_copy(
                tab_hbm.at[pl.ds(start * D, JC * D)], tbs[b], tsems[b]).start()

        def twait(b):
            pltpu.make_async_copy(
                tab_hbm.at[pl.ds(0, JC * D)], tbs[b], tsems[b]).wait()

        def ostart(i, j0, b):
            pltpu.make_async_copy(
                obs[b], out_hbm.at[i, pl.ds(j0, JC)], osems[b]).start()

        def owait(b):
            pltpu.make_async_copy(
                out_hbm.at[0, pl.ds(0, JC)], obs[b], osems[b]).wait()

        def compute_pair(ba, bb):
            tba, tbb = tbs[ba], tbs[bb]
            oba, obb = obs[ba], obs[bb]

            @plsc.parallel_loop(0, JC, unroll=4)
            def _(jj):
                rj = (JC - 1 - jj) * D
                for v in range(VPR):
                    sl = pl.ds(v * _LANES, _LANES)
                    xv = xbuf[jj, sl]
                    tsl = pl.ds(rj + v * _LANES, _LANES)
                    oba[jj, sl] = xv + tba[tsl]
                    obb[jj, sl] = xv + tbb[tsl]

        def jloop(jc, _):
            j0 = jc * JC
            pltpu.sync_copy(x_hbm.at[pl.ds(j0, JC)], xbuf)
            tstart(i0, j0, 0)
            tstart(i0 + 1, j0, 1)

            def tloop(tt, _):
                for p in range(2):
                    t = 2 * tt + p
                    r0 = i0 + 2 * t
                    ba, bb = 2 * p, 2 * p + 1

                    @pl.when(t < HALF - 1)
                    def _():
                        nb = 2 - 2 * p
                        tstart(r0 + 2, j0, nb)
                        tstart(r0 + 3, j0, nb + 1)

                    twait(ba)
                    twait(bb)

                    @pl.when(t >= 2)
                    def _():
                        owait(ba)
                        owait(bb)

                    compute_pair(ba, bb)
                    ostart(r0, j0, ba)
                    ostart(r0 + 1, j0, bb)
                return 0

            lax.fori_loop(0, QT, tloop, 0)
            for b in range(4):
                owait(b)
            return 0

        lax.fori_loop(0, NJC, jloop, 0)

    return sc_kernel


def kernel(x, rel_pos_embeddings):
    batch, S, D = x.shape
    T = rel_pos_embeddings.shape[0]
    info = plsc.get_sparse_core_info()
    sc = _build_sc_kernel(S, D, T, info.num_cores, info.num_subcores, 64)
    return sc(x.reshape(S, D), rel_pos_embeddings.reshape(T * D))


# speedup vs baseline: 10.6442x; 1.8170x over previous
import functools

import jax
import jax.numpy as jnp
from jax import lax
from jax.experimental import pallas as pl
from jax.experimental.pallas import tpu as pltpu
from jax.experimental.pallas import tpu_sc as plsc

_LANES = 16


@functools.lru_cache(maxsize=None)
def _build_sc_kernel(S, D, T, NC, NS, JC):
    NW = NC * NS
    ROWS = S // NW
    NJC = S // JC
    VPR = D // _LANES

    mesh = plsc.VectorSubcoreMesh(core_axis_name="c", subcore_axis_name="s")

    HALF = ROWS // 2
    QT = HALF // 2

    @functools.partial(
        pl.kernel,
        out_type=jax.ShapeDtypeStruct((S, S, D), jnp.float32),
        mesh=mesh,
        scratch_types=[
            pltpu.VMEM((JC, D), jnp.float32),
            [pltpu.VMEM((JC * D,), jnp.float32) for _ in range(4)],
            [pltpu.VMEM((JC, D), jnp.float32) for _ in range(4)],
            [pltpu.SemaphoreType.DMA for _ in range(8)],
            pltpu.VMEM_SHARED((T * D,), jnp.float32),
        ],
    )
    def sc_kernel(x_hbm, tab_hbm, out_hbm, xbuf, tbs, obs, sems, stab):
        tsems, osems = sems[:4], sems[4:]
        sid = lax.axis_index("s")
        wid = sid * NC + lax.axis_index("c")
        i0 = wid * ROWS
        ML = (T - 1) // 2

        @pl.when(sid == 0)
        def _():
            pltpu.sync_copy(tab_hbm, stab)

        plsc.subcore_barrier()

        def tstart(i, j0, b):
            start = i + (ML - JC + 1) - j0
            pltpu.make_async_copy(
                stab.at[pl.ds(start * D, JC * D)], tbs[b], tsems[b]).start()

        def twait(b):
            pltpu.make_async_copy(
                stab.at[pl.ds(0, JC * D)], tbs[b], tsems[b]).wait()

        def ostart(i, j0, b):
            pltpu.make_async_copy(
                obs[b], out_hbm.at[i, pl.ds(j0, JC)], osems[b]).start()

        def owait(b):
            pltpu.make_async_copy(
                out_hbm.at[0, pl.ds(0, JC)], obs[b], osems[b]).wait()

        def compute_pair(ba, bb):
            tba, tbb = tbs[ba], tbs[bb]
            oba, obb = obs[ba], obs[bb]

            @plsc.parallel_loop(0, JC, unroll=4)
            def _(jj):
                rj = (JC - 1 - jj) * D
                for v in range(VPR):
                    sl = pl.ds(v * _LANES, _LANES)
                    xv = xbuf[jj, sl]
                    tsl = pl.ds(rj + v * _LANES, _LANES)
                    oba[jj, sl] = xv + tba[tsl]
                    obb[jj, sl] = xv + tbb[tsl]

        def jloop(jc, _):
            j0 = jc * JC
            pltpu.sync_copy(x_hbm.at[pl.ds(j0, JC)], xbuf)
            tstart(i0, j0, 0)
            tstart(i0 + 1, j0, 1)

            def tloop(tt, _):
                for p in range(2):
                    t = 2 * tt + p
                    r0 = i0 + 2 * t
                    ba, bb = 2 * p, 2 * p + 1

                    @pl.when(t < HALF - 1)
                    def _():
                        nb = 2 - 2 * p
                        tstart(r0 + 2, j0, nb)
                        tstart(r0 + 3, j0, nb + 1)

                    twait(ba)
                    twait(bb)

                    @pl.when(t >= 2)
                    def _():
                        owait(ba)
                        owait(bb)

                    compute_pair(ba, bb)
                    ostart(r0, j0, ba)
                    ostart(r0 + 1, j0, bb)
                return 0

            lax.fori_loop(0, QT, tloop, 0)
            for b in range(4):
                owait(b)
            return 0

        lax.fori_loop(0, NJC, jloop, 0)

    return sc_kernel


def kernel(x, rel_pos_embeddings):
    batch, S, D = x.shape
    T = rel_pos_embeddings.shape[0]
    info = plsc.get_sparse_core_info()
    sc = _build_sc_kernel(S, D, T, info.num_cores, info.num_subcores, 64)
    return sc(x.reshape(S, D), rel_pos_embeddings.reshape(T * D))


# speedup vs baseline: 11.0170x; 1.0350x over previous
import functools

import jax
import jax.numpy as jnp
from jax import lax
from jax.experimental import pallas as pl
from jax.experimental.pallas import tpu as pltpu
from jax.experimental.pallas import tpu_sc as plsc

_LANES = 16


@functools.lru_cache(maxsize=None)
def _build_sc_kernel(S, D, T, NC, NS, JC):
    NW = NC * NS
    ROWS = S // NW
    NJC = S // JC
    VPR = D // _LANES

    mesh = plsc.VectorSubcoreMesh(core_axis_name="c", subcore_axis_name="s")

    HALF = ROWS // 2
    QT = HALF // 2

    @functools.partial(
        pl.kernel,
        out_type=jax.ShapeDtypeStruct((S, S, D), jnp.float32),
        mesh=mesh,
        scratch_types=[
            pltpu.VMEM((JC, D), jnp.float32),
            [pltpu.VMEM((JC * D,), jnp.float32) for _ in range(4)],
            [pltpu.VMEM((JC, D), jnp.float32) for _ in range(4)],
            [pltpu.SemaphoreType.DMA for _ in range(8)],
            pltpu.VMEM_SHARED((T * D,), jnp.float32),
        ],
    )
    def sc_kernel(x_hbm, tab_hbm, out_hbm, xbuf, tbs, obs, sems, stab):
        tsems, osems = sems[:4], sems[4:]
        sid = lax.axis_index("s")
        wid = sid * NC + lax.axis_index("c")
        i0 = wid * ROWS
        ML = (T - 1) // 2

        @pl.when(sid == 0)
        def _():
            pltpu.sync_copy(tab_hbm, stab)

        plsc.subcore_barrier()

        def tstart(i, j0, b):
            start = i + (ML - JC + 1) - j0
            pltpu.make_async_copy(
                stab.at[pl.ds(start * D, JC * D)], tbs[b], tsems[b]).start()

        def twait(b):
            pltpu.make_async_copy(
                stab.at[pl.ds(0, JC * D)], tbs[b], tsems[b]).wait()

        def ostart(i, j0, b):
            pltpu.make_async_copy(
                obs[b], out_hbm.at[i, pl.ds(j0, JC)], osems[b]).start()

        def owait(b):
            pltpu.make_async_copy(
                out_hbm.at[0, pl.ds(0, JC)], obs[b], osems[b]).wait()

        def compute_pair(ba, bb):
            tba, tbb = tbs[ba], tbs[bb]
            oba, obb = obs[ba], obs[bb]

            @plsc.parallel_loop(0, JC, unroll=4)
            def _(jj):
                rj = (JC - 1 - jj) * D
                for v in range(VPR):
                    sl = pl.ds(v * _LANES, _LANES)
                    xv = xbuf[jj, sl]
                    tsl = pl.ds(rj + v * _LANES, _LANES)
                    oba[jj, sl] = xv + tba[tsl]
                    obb[jj, sl] = xv + tbb[tsl]

        NPAIR = NJC * HALF

        def pair_params(g):
            jc = g // HALF
            t = g - jc * HALF
            return i0 + 2 * t, jc * JC

        r0p, j0p = pair_params(0)
        tstart(r0p, j0p, 0)
        tstart(r0p + 1, j0p, 1)

        def gloop(g2, _):
            for p in range(2):
                g = 2 * g2 + p
                r0, j0 = pair_params(g)
                ba, bb = 2 * p, 2 * p + 1

                @pl.when(g % HALF == 0)
                def _():
                    pltpu.sync_copy(x_hbm.at[pl.ds(j0, JC)], xbuf)

                @pl.when(g < NPAIR - 1)
                def _():
                    nr0, nj0 = pair_params(g + 1)
                    nb = 2 - 2 * p
                    tstart(nr0, nj0, nb)
                    tstart(nr0 + 1, nj0, nb + 1)

                twait(ba)
                twait(bb)

                @pl.when(g >= 2)
                def _():
                    owait(ba)
                    owait(bb)

                compute_pair(ba, bb)
                ostart(r0, j0, ba)
                ostart(r0 + 1, j0, bb)
            return 0

        lax.fori_loop(0, NPAIR // 2, gloop, 0)
        for b in range(4):
            owait(b)

    return sc_kernel


def kernel(x, rel_pos_embeddings):
    batch, S, D = x.shape
    T = rel_pos_embeddings.shape[0]
    info = plsc.get_sparse_core_info()
    sc = _build_sc_kernel(S, D, T, info.num_cores, info.num_subcores, 64)
    return sc(x.reshape(S, D), rel_pos_embeddings.reshape(T * D))


# speedup vs baseline: 12.0876x; 1.0972x over previous
import functools

import jax
import jax.numpy as jnp
from jax import lax
from jax.experimental import pallas as pl
from jax.experimental.pallas import tpu as pltpu
from jax.experimental.pallas import tpu_sc as plsc

_LANES = 16


@functools.lru_cache(maxsize=None)
def _build_sc_kernel(S, D, T, NC, NS, JC):
    NW = NC * NS
    ROWS = S // NW
    NJC = S // JC
    VPR = D // _LANES

    mesh = plsc.VectorSubcoreMesh(core_axis_name="c", subcore_axis_name="s")

    Q = 4
    NGJ = ROWS // Q
    NG = NJC * NGJ
    TR = JC + Q - 1

    @functools.partial(
        pl.kernel,
        out_type=jax.ShapeDtypeStruct((S, S, D), jnp.float32),
        mesh=mesh,
        scratch_types=[
            pltpu.VMEM((JC, D), jnp.float32),
            [pltpu.VMEM((TR * D,), jnp.float32) for _ in range(2)],
            [pltpu.VMEM((JC, D), jnp.float32) for _ in range(2 * Q)],
            [pltpu.SemaphoreType.DMA for _ in range(2)],
            [pltpu.SemaphoreType.DMA for _ in range(2 * Q)],
            pltpu.VMEM_SHARED((T * D,), jnp.float32),
        ],
    )
    def sc_kernel(x_hbm, tab_hbm, out_hbm, xbuf, tbs, obs, tsems, osems, stab):
        sid = lax.axis_index("s")
        wid = sid * NC + lax.axis_index("c")
        i0 = wid * ROWS
        ML = (T - 1) // 2

        @pl.when(sid == 0)
        def _():
            pltpu.sync_copy(tab_hbm, stab)

        plsc.subcore_barrier()

        def tstart(r0, j0, b):
            start = r0 + (ML - JC + 1) - j0
            pltpu.make_async_copy(
                stab.at[pl.ds(start * D, TR * D)], tbs[b], tsems[b]).start()

        def twait(b):
            pltpu.make_async_copy(
                stab.at[pl.ds(0, TR * D)], tbs[b], tsems[b]).wait()

        def ostart(i, j0, b):
            pltpu.make_async_copy(
                obs[b], out_hbm.at[i, pl.ds(j0, JC)], osems[b]).start()

        def owait(b):
            pltpu.make_async_copy(
                out_hbm.at[0, pl.ds(0, JC)], obs[b], osems[b]).wait()

        def compute_group(p):
            tb = tbs[p]
            og = obs[Q * p:Q * p + Q]

            @plsc.parallel_loop(0, JC, unroll=4)
            def _(jj):
                base = (JC - 1 - jj) * D
                for v in range(VPR):
                    sl = pl.ds(v * _LANES, _LANES)
                    xv = xbuf[jj, sl]
                    for q in range(Q):
                        og[q][jj, sl] = xv + tb[pl.ds(base + q * D + v * _LANES, _LANES)]

        def group_params(g):
            jc = g // NGJ
            t = g - jc * NGJ
            return i0 + Q * t, jc * JC

        r0p, j0p = group_params(0)
        tstart(r0p, j0p, 0)

        def gloop(g2, _):
            for p in range(2):
                g = 2 * g2 + p
                r0, j0 = group_params(g)

                @pl.when(g % NGJ == 0)
                def _():
                    pltpu.sync_copy(x_hbm.at[pl.ds(j0, JC)], xbuf)

                @pl.when(g < NG - 1)
                def _():
                    nr0, nj0 = group_params(g + 1)
                    tstart(nr0, nj0, 1 - p)

                twait(p)

                @pl.when(g >= 2)
                def _():
                    for q in range(Q):
                        owait(Q * p + q)

                compute_group(p)
                for q in range(Q):
                    ostart(r0 + q, j0, Q * p + q)
            return 0

        lax.fori_loop(0, NG // 2, gloop, 0)
        for b in range(2 * Q):
            owait(b)

    return sc_kernel


def kernel(x, rel_pos_embeddings):
    batch, S, D = x.shape
    T = rel_pos_embeddings.shape[0]
    info = plsc.get_sparse_core_info()
    sc = _build_sc_kernel(S, D, T, info.num_cores, info.num_subcores, 64)
    return sc(x.reshape(S, D), rel_pos_embeddings.reshape(T * D))


# speedup vs baseline: 12.6949x; 1.0502x over previous
import functools

import jax
import jax.numpy as jnp
from jax import lax
from jax.experimental import pallas as pl
from jax.experimental.pallas import tpu as pltpu
from jax.experimental.pallas import tpu_sc as plsc

_LANES = 16


@functools.lru_cache(maxsize=None)
def _build_sc_kernel(S, D, T, NC, NS, JC):
    NW = NC * NS
    ROWS = S // NW
    NJC = S // JC
    VPR = D // _LANES

    mesh = plsc.VectorSubcoreMesh(core_axis_name="c", subcore_axis_name="s")

    Q = 4
    NGJ = ROWS // Q
    NG = NJC * NGJ
    TR = JC + Q - 1

    @functools.partial(
        pl.kernel,
        out_type=jax.ShapeDtypeStruct((S, S, D), jnp.float32),
        mesh=mesh,
        scratch_types=[
            pltpu.VMEM((JC, D), jnp.float32),
            [pltpu.VMEM((TR * D,), jnp.float32) for _ in range(2)],
            [pltpu.VMEM((Q, JC, D), jnp.float32) for _ in range(2)],
            [pltpu.SemaphoreType.DMA for _ in range(2)],
            [pltpu.SemaphoreType.DMA for _ in range(2)],
            pltpu.VMEM_SHARED((T * D,), jnp.float32),
        ],
    )
    def sc_kernel(x_hbm, tab_hbm, out_hbm, xbuf, tbs, obs, tsems, osems, stab):
        sid = lax.axis_index("s")
        wid = sid * NC + lax.axis_index("c")
        i0 = wid * ROWS
        ML = (T - 1) // 2

        @pl.when(sid == 0)
        def _():
            pltpu.sync_copy(tab_hbm, stab)

        plsc.subcore_barrier()

        def tstart(r0, j0, b):
            start = r0 + (ML - JC + 1) - j0
            pltpu.make_async_copy(
                stab.at[pl.ds(start * D, TR * D)], tbs[b], tsems[b]).start()

        def twait(b):
            pltpu.make_async_copy(
                stab.at[pl.ds(0, TR * D)], tbs[b], tsems[b]).wait()

        def ostart(r0, j0, b):
            pltpu.make_async_copy(
                obs[b], out_hbm.at[pl.ds(r0, Q), pl.ds(j0, JC)], osems[b]).start()

        def owait(b):
            pltpu.make_async_copy(
                out_hbm.at[pl.ds(0, Q), pl.ds(0, JC)], obs[b], osems[b]).wait()

        def compute_group(p):
            tb = tbs[p]
            og = obs[p]

            @plsc.parallel_loop(0, JC, unroll=4)
            def _(jj):
                base = (JC - 1 - jj) * D
                for v in range(VPR):
                    sl = pl.ds(v * _LANES, _LANES)
                    xv = xbuf[jj, sl]
                    for q in range(Q):
                        og[q, jj, sl] = xv + tb[pl.ds(base + q * D + v * _LANES, _LANES)]

        def group_params(g):
            jc = g // NGJ
            t = g - jc * NGJ
            return i0 + Q * t, jc * JC

        r0p, j0p = group_params(0)
        tstart(r0p, j0p, 0)

        def gloop(g2, _):
            for p in range(2):
                g = 2 * g2 + p
                r0, j0 = group_params(g)

                @pl.when(g % NGJ == 0)
                def _():
                    pltpu.sync_copy(x_hbm.at[pl.ds(j0, JC)], xbuf)

                @pl.when(g < NG - 1)
                def _():
                    nr0, nj0 = group_params(g + 1)
                    tstart(nr0, nj0, 1 - p)

                twait(p)

                @pl.when(g >= 2)
                def _():
                    owait(p)

                compute_group(p)
                ostart(r0, j0, p)
            return 0

        lax.fori_loop(0, NG // 2, gloop, 0)
        for b in range(2):
            owait(b)

    return sc_kernel


def kernel(x, rel_pos_embeddings):
    batch, S, D = x.shape
    T = rel_pos_embeddings.shape[0]
    info = plsc.get_sparse_core_info()
    sc = _build_sc_kernel(S, D, T, info.num_cores, info.num_subcores, 64)
    return sc(x.reshape(S, D), rel_pos_embeddings.reshape(T * D))
